# baseline (speedup 1.0000x reference)
import functools

import jax
import jax.numpy as jnp
from jax import lax
from jax.experimental import pallas as pl
from jax.experimental.pallas import tpu as pltpu
from jax.experimental.pallas import tpu_sc as plsc

VOCAB = 1000000
EMBED_DIM = 64
WIDE = 128
BATCH = 4096
HIST = 200
TOT = BATCH * HIST

_info = plsc.get_sparse_core_info()
_NCORES = _info.num_cores
_NSUB = _info.num_subcores
NW = _NCORES * _NSUB
BTILE = BATCH // NW
NBUF = 2

_mesh = plsc.VectorSubcoreMesh(core_axis_name="c", subcore_axis_name="s")

_ONE16 = None


@functools.partial(
    pl.kernel,
    mesh=_mesh,
    out_type=jax.ShapeDtypeStruct((HIST, 8, NW, 8, WIDE), jnp.float32),
    scratch_types=[
        pltpu.VMEM((HIST, BTILE), jnp.int32),
        pltpu.VMEM((NBUF, BTILE), jnp.int32),
        pltpu.VMEM((NBUF, BTILE, WIDE), jnp.float32),
        pltpu.VMEM((NBUF, EMBED_DIM, BTILE), jnp.float32),
        pltpu.SemaphoreType.DMA((NBUF,)),
        pltpu.SemaphoreType.DMA((NBUF,)),
        pltpu.SemaphoreType.DMA,
    ],
    compiler_params=pltpu.CompilerParams(needs_layout_passes=False),
)
def _sc_gather(idx_hbm, table_hbm, out_hbm, idx_v, gidx_v, prow_v, obuf_v,
               gsem, ssem, isem):
    wid = lax.axis_index("s") * _NCORES + lax.axis_index("c")

    pltpu.async_copy(
        idx_hbm.at[pl.ds(0, HIST), pl.ds(wid * BTILE, BTILE)], idx_v, isem
    ).wait()

    lanes = lax.iota(jnp.int32, 16)
    one16 = jnp.full((16,), 1, jnp.int32)
    d16 = jnp.full((16,), EMBED_DIM, jnp.int32)

    def issue_gather(h, slot):
        for g in range(BTILE // 16):
            v16 = plsc.load_gather(idx_v, [jnp.full((16,), 0, jnp.int32) + h,
                                           g * 16 + lanes])
            gidx_v[slot, pl.ds(g * 16, 16)] = lax.shift_right_arithmetic(
                v16, one16)
        pltpu.async_copy(
            table_hbm.at[gidx_v.at[slot]], prow_v.at[slot], gsem.at[slot])

    def gather_wait(slot):
        pltpu.make_async_copy(
            table_hbm.at[pl.ds(0, BTILE)], prow_v.at[slot], gsem.at[slot]
        ).wait()

    def select_transpose(h, slot):
        for g in range(BTILE // 16):
            rows16 = g * 16 + lanes
            v16 = plsc.load_gather(idx_v, [jnp.full((16,), 0, jnp.int32) + h,
                                           rows16])
            col0 = lax.mul(lax.bitwise_and(v16, one16), d16)
            for e in range(EMBED_DIM):
                vals = plsc.load_gather(
                    prow_v.at[slot],
                    [rows16, col0 + e])
                obuf_v[slot, e, pl.ds(g * 16, 16)] = vals

    def issue_store(h, slot):
        for e8 in range(8):
            pltpu.async_copy(
                obuf_v.at[slot, pl.ds(e8 * 8, 8)],
                out_hbm.at[h, e8, wid],
                ssem.at[slot])

    def store_wait(slot):
        for e8 in range(8):
            pltpu.make_async_copy(
                table_hbm.at[pl.ds(0, 8)],
                obuf_v.at[slot, pl.ds(e8 * 8, 8)],
                ssem.at[slot],
            ).wait()

    def visit(h, slot, nh, nslot, guard_gather, guard_store):
        if nh is not None:
            if guard_gather is None:
                issue_gather(nh, nslot)
            else:
                @pl.when(guard_gather)
                def _():
                    issue_gather(nh, nslot)
        gather_wait(slot)
        @pl.when(guard_store)
        def _():
            store_wait(slot)
        select_transpose(h, slot)
        issue_store(h, slot)

    issue_gather(0, 0)

    def body(m, carry):
        h0 = 2 * m
        visit(h0, 0, h0 + 1, 1, None, m > 0)
        visit(h0 + 1, 1, h0 + 2, 0, m < 99, m > 0)
        return carry

    lax.fori_loop(0, 100, body, 0)
    store_wait(0)
    store_wait(1)


def kernel(input, table):
    idx_t = input.astype(jnp.int32).T
    t2 = table.reshape(VOCAB // 2, WIDE)
    out6 = _sc_gather(idx_t, t2)
    out = out6.transpose(2, 4, 0, 1, 3).reshape(BATCH, HIST, EMBED_DIM)
    return out


# speedup vs baseline: 1.8538x; 1.8538x over previous
import functools

import jax
import jax.numpy as jnp
from jax import lax
from jax.experimental import pallas as pl
from jax.experimental.pallas import tpu as pltpu
from jax.experimental.pallas import tpu_sc as plsc

VOCAB = 1000000
EMBED_DIM = 64
WIDE = 128
BATCH = 4096
HIST = 200
TOT = BATCH * HIST

_info = plsc.get_sparse_core_info()
_NCORES = _info.num_cores
_NSUB = _info.num_subcores
NW = _NCORES * _NSUB
PER = TOT // NW
CHUNK = 256
NSTEPS = PER // CHUNK
NBUF = 2
LOOKAHEAD = 1

_mesh = plsc.VectorSubcoreMesh(core_axis_name="c", subcore_axis_name="s")


@functools.partial(
    pl.kernel,
    mesh=_mesh,
    out_type=jax.ShapeDtypeStruct((TOT, WIDE), jnp.float32),
    scratch_types=[
        pltpu.VMEM((PER,), jnp.int32),
        pltpu.VMEM((NBUF, CHUNK, WIDE), jnp.float32),
        pltpu.SemaphoreType.DMA((NBUF,)),
        pltpu.SemaphoreType.DMA((NBUF,)),
        pltpu.SemaphoreType.DMA,
    ],
)
def _sc_gather(idx_hbm, table_hbm, out_hbm, idx_v, rows_v, gsem, ssem, isem):
    wid = lax.axis_index("s") * _NCORES + lax.axis_index("c")
    base = wid * PER

    pltpu.async_copy(idx_hbm.at[pl.ds(base, PER)], idx_v, isem).wait()

    def issue_gather(k, slot):
        pltpu.async_copy(
            table_hbm.at[idx_v.at[pl.ds(k * CHUNK, CHUNK)]],
            rows_v.at[slot],
            gsem.at[slot],
        )

    def gather_wait(slot):
        pltpu.make_async_copy(
            table_hbm.at[pl.ds(0, CHUNK)], rows_v.at[slot], gsem.at[slot]
        ).wait()

    def issue_store(k, slot):
        pltpu.async_copy(
            rows_v.at[slot], out_hbm.at[pl.ds(base + k * CHUNK, CHUNK)],
            ssem.at[slot],
        )

    def store_wait(slot):
        pltpu.make_async_copy(
            table_hbm.at[pl.ds(0, CHUNK)], rows_v.at[slot], ssem.at[slot]
        ).wait()

    for k in range(LOOKAHEAD):
        issue_gather(k, k % NBUF)
    for k in range(LOOKAHEAD):
        slot = k % NBUF
        gather_wait(slot)
        issue_store(k, slot)
        issue_gather(k + LOOKAHEAD, (k + LOOKAHEAD) % NBUF)

    n_main = NSTEPS - 2 * LOOKAHEAD
    assert n_main % NBUF == 0

    def outer(m, carry):
        k0 = LOOKAHEAD + m * NBUF
        for b in range(NBUF):
            slot = (LOOKAHEAD + b) % NBUF
            k = k0 + b
            gather_wait(slot)
            issue_store(k, slot)
            nslot = b
            store_wait(nslot)
            issue_gather(k + LOOKAHEAD, nslot)
        return carry

    lax.fori_loop(0, n_main // NBUF, outer, 0)

    for k in range(NSTEPS - LOOKAHEAD, NSTEPS):
        slot = k % NBUF
        gather_wait(slot)
        issue_store(k, slot)

    for b in range(NBUF):
        store_wait(b)


def kernel(input, table):
    idx = input.reshape(TOT).astype(jnp.int32)
    wide = jnp.concatenate(
        [table, jnp.zeros((VOCAB, WIDE - EMBED_DIM), jnp.float32)], axis=1
    )
    out = _sc_gather(idx, wide)
    return out[:, :EMBED_DIM].reshape(BATCH, HIST, EMBED_DIM)


# speedup vs baseline: 1.8550x; 1.0006x over previous
import functools

import jax
import jax.numpy as jnp
from jax import lax
from jax.experimental import pallas as pl
from jax.experimental.pallas import tpu as pltpu
from jax.experimental.pallas import tpu_sc as plsc

VOCAB = 1000000
EMBED_DIM = 64
WIDE = 128
BATCH = 4096
HIST = 200
TOT = BATCH * HIST

_info = plsc.get_sparse_core_info()
_NCORES = _info.num_cores
_NSUB = _info.num_subcores
NW = _NCORES * _NSUB
PER = TOT // NW
CHUNK = 256
NSTEPS = PER // CHUNK
NBUF = 3
LOOKAHEAD = 2

_mesh = plsc.VectorSubcoreMesh(core_axis_name="c", subcore_axis_name="s")


@functools.partial(
    pl.kernel,
    mesh=_mesh,
    out_type=jax.ShapeDtypeStruct((TOT, WIDE), jnp.float32),
    scratch_types=[
        pltpu.VMEM((PER,), jnp.int32),
        pltpu.VMEM((NBUF, CHUNK, WIDE), jnp.float32),
        pltpu.SemaphoreType.DMA((NBUF,)),
        pltpu.SemaphoreType.DMA((NBUF,)),
        pltpu.SemaphoreType.DMA,
    ],
)
def _sc_gather(idx_hbm, table_hbm, out_hbm, idx_v, rows_v, gsem, ssem, isem):
    wid = lax.axis_index("s") * _NCORES + lax.axis_index("c")
    base = wid * PER

    pltpu.async_copy(idx_hbm.at[pl.ds(base, PER)], idx_v, isem).wait()

    def issue_gather(k, slot):
        pltpu.async_copy(
            table_hbm.at[idx_v.at[pl.ds(k * CHUNK, CHUNK)]],
            rows_v.at[slot],
            gsem.at[slot],
        )

    def gather_wait(slot):
        pltpu.make_async_copy(
            table_hbm.at[pl.ds(0, CHUNK)], rows_v.at[slot], gsem.at[slot]
        ).wait()

    def issue_store(k, slot):
        pltpu.async_copy(
            rows_v.at[slot], out_hbm.at[pl.ds(base + k * CHUNK, CHUNK)],
            ssem.at[slot],
        )

    def store_wait(slot):
        pltpu.make_async_copy(
            table_hbm.at[pl.ds(0, CHUNK)], rows_v.at[slot], ssem.at[slot]
        ).wait()

    for k in range(LOOKAHEAD):
        issue_gather(k, k % NBUF)
    for k in range(LOOKAHEAD):
        slot = k % NBUF
        gather_wait(slot)
        issue_store(k, slot)
        issue_gather(k + LOOKAHEAD, (k + LOOKAHEAD) % NBUF)

    n_main = NSTEPS - 2 * LOOKAHEAD
    assert n_main % NBUF == 0

    def outer(m, carry):
        k0 = LOOKAHEAD + m * NBUF
        for b in range(NBUF):
            slot = (LOOKAHEAD + b) % NBUF
            k = k0 + b
            gather_wait(slot)
            issue_store(k, slot)
            nslot = (2 * LOOKAHEAD + b) % NBUF
            store_wait(nslot)
            issue_gather(k + LOOKAHEAD, nslot)
        return carry

    lax.fori_loop(0, n_main // NBUF, outer, 0)

    for k in range(NSTEPS - LOOKAHEAD, NSTEPS):
        slot = k % NBUF
        gather_wait(slot)
        issue_store(k, slot)

    for b in range(NBUF):
        store_wait(b)


def kernel(input, table):
    idx = input.reshape(TOT).astype(jnp.int32)
    wide = jnp.concatenate(
        [table, jnp.zeros((VOCAB, WIDE - EMBED_DIM), jnp.float32)], axis=1
    )
    out = _sc_gather(idx, wide)
    return out[:, :EMBED_DIM].reshape(BATCH, HIST, EMBED_DIM)


# speedup vs baseline: 1.8621x; 1.0038x over previous
import functools

import jax
import jax.numpy as jnp
from jax import lax
from jax.experimental import pallas as pl
from jax.experimental.pallas import tpu as pltpu
from jax.experimental.pallas import tpu_sc as plsc

VOCAB = 1000000
EMBED_DIM = 64
WIDE = 128
BATCH = 4096
HIST = 200
TOT = BATCH * HIST

_info = plsc.get_sparse_core_info()
_NCORES = _info.num_cores
_NSUB = _info.num_subcores
NW = _NCORES * _NSUB
PER = TOT // NW
CHUNK = 128
NSTEPS = PER // CHUNK
NBUF = 4
LOOKAHEAD = 2

_mesh = plsc.VectorSubcoreMesh(core_axis_name="c", subcore_axis_name="s")


@functools.partial(
    pl.kernel,
    mesh=_mesh,
    out_type=jax.ShapeDtypeStruct((TOT, WIDE), jnp.float32),
    scratch_types=[
        pltpu.VMEM((PER,), jnp.int32),
        pltpu.VMEM((NBUF, CHUNK, WIDE), jnp.float32),
        pltpu.SemaphoreType.DMA((NBUF,)),
        pltpu.SemaphoreType.DMA((NBUF,)),
        pltpu.SemaphoreType.DMA,
    ],
)
def _sc_gather(idx_hbm, table_hbm, out_hbm, idx_v, rows_v, gsem, ssem, isem):
    wid = lax.axis_index("s") * _NCORES + lax.axis_index("c")
    base = wid * PER

    pltpu.async_copy(idx_hbm.at[pl.ds(base, PER)], idx_v, isem).wait()

    def issue_gather(k, slot):
        pltpu.async_copy(
            table_hbm.at[idx_v.at[pl.ds(k * CHUNK, CHUNK)]],
            rows_v.at[slot],
            gsem.at[slot],
        )

    def gather_wait(slot):
        pltpu.make_async_copy(
            table_hbm.at[pl.ds(0, CHUNK)], rows_v.at[slot], gsem.at[slot]
        ).wait()

    def issue_store(k, slot):
        pltpu.async_copy(
            rows_v.at[slot], out_hbm.at[pl.ds(base + k * CHUNK, CHUNK)],
            ssem.at[slot],
        )

    def store_wait(slot):
        pltpu.make_async_copy(
            table_hbm.at[pl.ds(0, CHUNK)], rows_v.at[slot], ssem.at[slot]
        ).wait()

    for k in range(LOOKAHEAD):
        issue_gather(k, k % NBUF)
    for k in range(LOOKAHEAD):
        slot = k % NBUF
        gather_wait(slot)
        issue_store(k, slot)
        issue_gather(k + LOOKAHEAD, (k + LOOKAHEAD) % NBUF)

    n_main = NSTEPS - 2 * LOOKAHEAD
    assert n_main % NBUF == 0

    def outer(m, carry):
        k0 = LOOKAHEAD + m * NBUF
        for b in range(NBUF):
            slot = (LOOKAHEAD + b) % NBUF
            k = k0 + b
            gather_wait(slot)
            issue_store(k, slot)
            nslot = b
            store_wait(nslot)
            issue_gather(k + LOOKAHEAD, nslot)
        return carry

    lax.fori_loop(0, n_main // NBUF, outer, 0)

    for k in range(NSTEPS - LOOKAHEAD, NSTEPS):
        slot = k % NBUF
        gather_wait(slot)
        issue_store(k, slot)

    for b in range(NBUF):
        store_wait(b)


def kernel(input, table):
    idx = input.reshape(TOT).astype(jnp.int32)
    wide = jnp.concatenate(
        [table, jnp.zeros((VOCAB, WIDE - EMBED_DIM), jnp.float32)], axis=1
    )
    out = _sc_gather(idx, wide)
    return out[:, :EMBED_DIM].reshape(BATCH, HIST, EMBED_DIM)
